# baseline (device time: 57775 ns/iter reference)
import jax
import jax.numpy as jnp
from jax import lax
from jax.experimental import pallas as pl
from jax.experimental.pallas import tpu as pltpu

N_DEV = 4
N_EXP = 16
E_LOCAL = 4
N_TOK = 2048
D_IN = 512
D_OUT = 1024
CHUNK = N_TOK // N_DEV
HALF = D_OUT // 2
N_SUB = 4
SUB = CHUNK // N_SUB
BF16 = jnp.bfloat16
F32 = jnp.float32


def kernel(x, router_W, route_idx, expert_W):
    def body(x_ref, rw_ref, idx_ref, ew_ref, out_ref,
             sbuf_cw, rbuf_cw, agbuf_cw, sbuf_ccw, rbuf_ccw, agbuf_ccw,
             w_ref, xb_ref, ewb_ref, ss_cw, rs_cw, ss_ccw, rs_ccw):
        p = lax.axis_index("i")
        left = lax.rem(p + N_DEV - 1, N_DEV)
        right = lax.rem(p + 1, N_DEV)

        barrier_sem = pltpu.get_barrier_semaphore()
        for nbr in (left, right):
            pl.semaphore_signal(
                barrier_sem, inc=1,
                device_id=(nbr,), device_id_type=pl.DeviceIdType.MESH,
            )
        pl.semaphore_wait(barrier_sem, 2)

        scores = jnp.dot(x_ref[:, :], rw_ref[:, :],
                         preferred_element_type=F32)
        m = jnp.max(scores, axis=1, keepdims=True)
        ex = jnp.exp(scores - m)
        probs = ex / jnp.sum(ex, axis=1, keepdims=True)
        idx0 = idx_ref[:, 0:1]
        idx1 = idx_ref[:, 1:2]
        iota = lax.broadcasted_iota(jnp.int32, (N_TOK, N_EXP), 1)
        g0 = jnp.sum(jnp.where(iota == idx0, probs, 0.0), axis=1, keepdims=True)
        g1 = jnp.sum(jnp.where(iota == idx1, probs, 0.0), axis=1, keepdims=True)
        gs = g0 + g1
        g0 = g0 / gs
        g1 = g1 / gs
        eids = (p * E_LOCAL
                + lax.broadcasted_iota(jnp.int32, (N_TOK, E_LOCAL), 1))
        w_ref[:, :] = (jnp.where(idx0 == eids, g0, 0.0)
                       + jnp.where(idx1 == eids, g1, 0.0)).astype(BF16)
        xb_ref[:, :] = x_ref[:, :].astype(BF16)
        for j in range(E_LOCAL):
            ewb_ref[j] = ew_ref[j].astype(BF16)

        def rs_rdma(h, s, dirn):
            sbuf, rbuf, ss, rs, dev = (
                (sbuf_cw, rbuf_cw, ss_cw, rs_cw, right) if dirn == 0
                else (sbuf_ccw, rbuf_ccw, ss_ccw, rs_ccw, left))
            k = h * N_SUB + s
            return pltpu.make_async_remote_copy(
                src_ref=sbuf.at[h, s], dst_ref=rbuf.at[h, s],
                send_sem=ss.at[k], recv_sem=rs.at[k],
                device_id=(dev,), device_id_type=pl.DeviceIdType.MESH,
            )

        def ag_rdma(h, s, dirn):
            agbuf, ss, rs, dev = (
                (agbuf_cw, ss_cw, rs_cw, right) if dirn == 0
                else (agbuf_ccw, ss_ccw, rs_ccw, left))
            k = (N_DEV - 1 + h) * N_SUB + s
            return pltpu.make_async_remote_copy(
                src_ref=agbuf.at[h, s], dst_ref=agbuf.at[h + 1, s],
                send_sem=ss.at[k], recv_sem=rs.at[k],
                device_id=(dev,), device_id_type=pl.DeviceIdType.MESH,
            )

        def compute_chunk(c, store=True):
            rows = pl.ds(c * CHUNK, CHUNK)
            xa = xb_ref[rows, :]
            acc = None
            for j in range(E_LOCAL):
                t = jnp.dot(xa * w_ref[rows, j:j + 1], ewb_ref[j],
                            preferred_element_type=F32)
                acc = t if acc is None else acc + t
            if store:
                out_ref[rows, :] = acc
            return acc

        hop0 = []
        acc = compute_chunk(p, store=False)
        for s in range(N_SUB):
            r0, r1 = s * SUB, (s + 1) * SUB
            sbuf_cw[0, s] = acc[r0:r1, 0:HALF].astype(BF16)
            sbuf_ccw[0, s] = acc[r0:r1, HALF:D_OUT].astype(BF16)
            for dirn in (0, 1):
                rd = rs_rdma(0, s, dirn)
                rd.start()
                hop0.append(rd)
        compute_chunk(lax.rem(p + 1, N_DEV))
        compute_chunk(lax.rem(p + 3, N_DEV))

        rs_descs = {(0, s, d): hop0[s * 2 + d] for s in range(N_SUB)
                    for d in (0, 1)}
        for h in range(N_DEV - 1):
            if h == 1:
                compute_chunk(lax.rem(p + 2, N_DEV))
            rc_cw = lax.rem(p - h - 1 + N_DEV, N_DEV)
            rc_ccw = lax.rem(p + h + 1, N_DEV)
            for s in range(N_SUB):
                for dirn in (0, 1):
                    rc = rc_cw if dirn == 0 else rc_ccw
                    cols = slice(0, HALF) if dirn == 0 else slice(HALF, D_OUT)
                    rbuf = rbuf_cw if dirn == 0 else rbuf_ccw
                    sbuf = sbuf_cw if dirn == 0 else sbuf_ccw
                    agbuf = agbuf_cw if dirn == 0 else agbuf_ccw
                    rs_descs[(h, s, dirn)].wait()
                    rows = pl.ds(rc * CHUNK + s * SUB, SUB)
                    ssum = out_ref[rows, cols] + rbuf[h, s].astype(F32)
                    if h < N_DEV - 2:
                        sbuf[h + 1, s] = ssum.astype(BF16)
                        rd = rs_rdma(h + 1, s, dirn)
                        rd.start()
                        rs_descs[(h + 1, s, dirn)] = rd
                    else:
                        agbuf[0, s] = ssum.astype(BF16)
                        rd = ag_rdma(0, s, dirn)
                        rd.start()
                        rs_descs[("ag", s, dirn)] = rd
                        out_ref[rows, cols] = ssum

        ag_descs = {(0, s, d): rs_descs[("ag", s, d)] for s in range(N_SUB)
                    for d in (0, 1)}
        for h in range(N_DEV - 1):
            rc_cw = lax.rem(p - h + N_DEV, N_DEV)
            rc_ccw = lax.rem(p + h, N_DEV)
            for s in range(N_SUB):
                for dirn in (0, 1):
                    rc = rc_cw if dirn == 0 else rc_ccw
                    cols = slice(0, HALF) if dirn == 0 else slice(HALF, D_OUT)
                    agbuf = agbuf_cw if dirn == 0 else agbuf_ccw
                    ag_descs[(h, s, dirn)].wait()
                    if h < N_DEV - 2:
                        rd = ag_rdma(h + 1, s, dirn)
                        rd.start()
                        ag_descs[(h + 1, s, dirn)] = rd
                    rows = pl.ds(rc * CHUNK + s * SUB, SUB)
                    out_ref[rows, cols] = agbuf[h + 1, s].astype(F32)

    n_sems = 2 * (N_DEV - 1) * N_SUB
    return pl.pallas_call(
        body,
        out_shape=jax.ShapeDtypeStruct((N_TOK, D_OUT), F32),
        in_specs=[
            pl.BlockSpec(memory_space=pltpu.VMEM),
            pl.BlockSpec(memory_space=pltpu.VMEM),
            pl.BlockSpec(memory_space=pltpu.VMEM),
            pl.BlockSpec(memory_space=pltpu.VMEM),
        ],
        out_specs=pl.BlockSpec(memory_space=pltpu.VMEM),
        scratch_shapes=[
            pltpu.VMEM((N_DEV - 1, N_SUB, SUB, HALF), BF16),
            pltpu.VMEM((N_DEV - 1, N_SUB, SUB, HALF), BF16),
            pltpu.VMEM((N_DEV, N_SUB, SUB, HALF), BF16),
            pltpu.VMEM((N_DEV - 1, N_SUB, SUB, HALF), BF16),
            pltpu.VMEM((N_DEV - 1, N_SUB, SUB, HALF), BF16),
            pltpu.VMEM((N_DEV, N_SUB, SUB, HALF), BF16),
            pltpu.VMEM((N_TOK, E_LOCAL), BF16),
            pltpu.VMEM((N_TOK, D_IN), BF16),
            pltpu.VMEM((E_LOCAL, D_IN, D_OUT), BF16),
            pltpu.SemaphoreType.DMA((n_sems,)),
            pltpu.SemaphoreType.DMA((n_sems,)),
            pltpu.SemaphoreType.DMA((n_sems,)),
            pltpu.SemaphoreType.DMA((n_sems,)),
        ],
        compiler_params=pltpu.CompilerParams(collective_id=0),
    )(x, router_W, route_idx, expert_W)
